# baseline (speedup 1.0000x reference)
import functools

import jax
import jax.numpy as jnp
from jax import lax
from jax.experimental import pallas as pl
from jax.experimental.pallas import tpu as pltpu
from jax.experimental.pallas import tpu_sc as plsc

BATCH = 16384
EMBED_DIM = 64
_NC = 2
_NS = 16
_NW = _NC * _NS
_BPW = BATCH // _NW
_CHUNK = 128
_NCHUNK = _BPW // _CHUNK
_NRING = 10
_AHEAD = 8


def _cf_body(su_hbm, si_hbm, fl_hbm, sl_hbm, ut_hbm, it_hbm, out_hbm,
             su_v, si_v, fl_v, sl_v, ipair, ublk, ibuf, out_v, usem, isem):
    wid = lax.axis_index("s") * _NC + lax.axis_index("c")
    base = wid * _BPW

    pltpu.sync_copy(su_hbm.at[pl.ds(base, _BPW)], su_v)
    pltpu.sync_copy(si_hbm.at[pl.ds(base, _BPW)], si_v)
    pltpu.sync_copy(fl_hbm.at[pl.ds(base, _BPW)], fl_v)
    pltpu.sync_copy(sl_hbm.at[pl.ds(base, _BPW)], sl_v)

    def iprep(j, b):
        for t in range(_CHUNK // 16):
            s = pl.ds(j * _CHUNK + t * 16, 16)
            ipair.at[pl.ds(b * _CHUNK + t * 16, 16)][...] = si_v[s] >> 1
        return pltpu.async_copy(
            it_hbm.at[ipair.at[pl.ds(b * _CHUNK, _CHUNK)]],
            ibuf.at[pl.ds(b * _CHUNK, _CHUNK), :], isem.at[b])

    def issue(uid, flag, slot):
        @pl.when(flag == 1)
        def _():
            blk = pl.multiple_of((uid >> 7) * 128, 128)
            row = pl.multiple_of(slot * EMBED_DIM, 8)
            pltpu.async_copy(ut_hbm.at[:, pl.ds(blk, 128)],
                             ublk.at[pl.ds(row, EMBED_DIM), :], usem.at[slot])

    def wait_for(flag, slot):
        @pl.when(flag == 1)
        def _():
            row = pl.multiple_of(slot * EMBED_DIM, 8)
            pltpu.make_async_copy(ut_hbm.at[:, pl.ds(0, 128)],
                                  ublk.at[pl.ds(row, EMBED_DIM), :],
                                  usem.at[slot]).wait()

    lanes = lax.iota(jnp.int32, 16)
    d16 = [lax.iota(jnp.int32, 16) + 16 * q for q in range(EMBED_DIM // 16)]

    uc0 = su_v[pl.ds(0, 16)]
    fl0 = fl_v[pl.ds(0, 16)]
    sl0 = sl_v[pl.ds(0, 16)]
    for k in range(_AHEAD):
        issue(uc0[k], fl0[k], sl0[k])

    inflight_i = iprep(0, 0)
    for j in range(_NCHUNK):
        b = j % 2
        cur_i = inflight_i
        if j + 1 < _NCHUNK:
            inflight_i = iprep(j + 1, 1 - b)
        cur_i.wait()

        def block16(g, _):
            rbase = j * _CHUNK + g * 16
            uc = su_v[pl.ds(rbase, 16)]
            flc = fl_v[pl.ds(rbase, 16)]
            slc = sl_v[pl.ds(rbase, 16)]
            ipar = (si_v[pl.ds(rbase, 16)] & 1) * EMBED_DIM
            nxt = jnp.minimum(rbase + 16, _BPW - 16)
            un = su_v[pl.ds(nxt, 16)]
            fln = fl_v[pl.ds(nxt, 16)]
            sln = sl_v[pl.ds(nxt, 16)]
            acc = jnp.zeros((16,), jnp.float32)
            for k in range(16):
                wait_for(flc[k], slc[k])
                lane = jnp.broadcast_to(uc[k] & 127, (16,))
                srow = slc[k] * EMBED_DIM
                psum = jnp.zeros((16,), jnp.float32)
                for q in range(EMBED_DIM // 16):
                    u = plsc.load_gather(ublk, [srow + d16[q], lane])
                    v = ibuf[b * _CHUNK + g * 16 + k,
                             pl.ds(ipar[k] + 16 * q, 16)]
                    psum = psum + u * v
                if k < _AHEAD:
                    issue(uc[k + _AHEAD], flc[k + _AHEAD], slc[k + _AHEAD])
                else:
                    kk = k - _AHEAD

                    @pl.when(rbase + 16 + kk < _BPW)
                    def _():
                        issue(un[kk], fln[kk], sln[kk])
                acc = jnp.where(lanes == k, jnp.sum(psum), acc)
            out_v[pl.ds(rbase, 16)] = acc
            return 0

        lax.fori_loop(0, _CHUNK // 16, block16, 0)

    pltpu.sync_copy(out_v, out_hbm.at[pl.ds(base, _BPW)])


@jax.jit
def _cf_kernel(user_id, item_id, user_table, item_table):
    mesh = plsc.VectorSubcoreMesh(core_axis_name="c", subcore_axis_name="s")
    f = pl.kernel(
        _cf_body,
        out_type=jax.ShapeDtypeStruct((BATCH,), jnp.float32),
        mesh=mesh,
        scratch_types=[
            pltpu.VMEM((_BPW,), jnp.int32),
            pltpu.VMEM((_BPW,), jnp.int32),
            pltpu.VMEM((_BPW,), jnp.int32),
            pltpu.VMEM((_BPW,), jnp.int32),
            pltpu.VMEM((2 * _CHUNK,), jnp.int32),
            pltpu.VMEM((_NRING * EMBED_DIM, 128), jnp.float32),
            pltpu.VMEM((2 * _CHUNK, 2 * EMBED_DIM), jnp.float32),
            pltpu.VMEM((_BPW,), jnp.float32),
            pltpu.SemaphoreType.DMA((_NRING,)),
            pltpu.SemaphoreType.DMA((2,)),
        ],
        compiler_params=pltpu.CompilerParams(
            needs_layout_passes=False, use_tc_tiling_on_sc=True),
    )
    order = jnp.argsort(user_id)
    su = user_id[order]
    si = item_id[order]
    blk = su >> 7
    prev = jnp.concatenate([blk[:1] - 1, blk[:-1]])
    pos = jnp.arange(BATCH, dtype=jnp.int32)
    flags = ((pos % _BPW == 0) | (blk != prev)).astype(jnp.int32)
    slots = (jnp.cumsum(flags) - 1).astype(jnp.int32) % _NRING
    it2 = item_table.reshape(item_table.shape[0] // 2, 2 * EMBED_DIM)
    out_sorted = f(su, si, flags, slots, user_table.T, it2)
    return jnp.zeros((BATCH,), jnp.float32).at[order].set(out_sorted)


def kernel(user_id, item_id, user_table, item_table):
    out = _cf_kernel(user_id, item_id, user_table, item_table)
    return out.reshape(BATCH, 1)


# speedup vs baseline: 1.2651x; 1.2651x over previous
import functools

import jax
import jax.numpy as jnp
from jax import lax
from jax.experimental import pallas as pl
from jax.experimental.pallas import tpu as pltpu
from jax.experimental.pallas import tpu_sc as plsc

BATCH = 16384
EMBED_DIM = 64
_NC = 2
_NS = 16
_NW = _NC * _NS
_BPW = BATCH // _NW
_CHUNK = 128
_NCHUNK = _BPW // _CHUNK
_NRING = 10
_AHEAD = 8


def _cf_body(su_hbm, si_hbm, fl_hbm, sl_hbm, ut_hbm, it_hbm, out_hbm,
             su_v, si_v, fl_v, sl_v, ipair, ublk, ibuf, out_v, usem, isem):
    wid = lax.axis_index("s") * _NC + lax.axis_index("c")
    base = wid * _BPW

    pltpu.sync_copy(su_hbm.at[pl.ds(base, _BPW)], su_v)
    pltpu.sync_copy(si_hbm.at[pl.ds(base, _BPW)], si_v)
    pltpu.sync_copy(fl_hbm.at[pl.ds(base, _BPW)], fl_v)
    pltpu.sync_copy(sl_hbm.at[pl.ds(base, _BPW)], sl_v)

    def iprep(j, b):
        for t in range(_CHUNK // 16):
            s = pl.ds(j * _CHUNK + t * 16, 16)
            ipair.at[pl.ds(b * _CHUNK + t * 16, 16)][...] = si_v[s] >> 1
        return pltpu.async_copy(
            it_hbm.at[ipair.at[pl.ds(b * _CHUNK, _CHUNK)]],
            ibuf.at[pl.ds(b * _CHUNK, _CHUNK), :], isem.at[b])

    def issue(uid, flag, slot):
        @pl.when(flag == 1)
        def _():
            blk = pl.multiple_of((uid >> 7) * 128, 128)
            row = pl.multiple_of(slot * EMBED_DIM, 8)
            pltpu.async_copy(ut_hbm.at[:, pl.ds(blk, 128)],
                             ublk.at[pl.ds(row, EMBED_DIM), :], usem.at[slot])

    def wait_for(flag, slot):
        @pl.when(flag == 1)
        def _():
            row = pl.multiple_of(slot * EMBED_DIM, 8)
            pltpu.make_async_copy(ut_hbm.at[:, pl.ds(0, 128)],
                                  ublk.at[pl.ds(row, EMBED_DIM), :],
                                  usem.at[slot]).wait()

    lanes = lax.iota(jnp.int32, 16)
    d16 = [lax.iota(jnp.int32, 16) + 16 * q for q in range(EMBED_DIM // 16)]

    uc0 = su_v[pl.ds(0, 16)]
    fl0 = fl_v[pl.ds(0, 16)]
    sl0 = sl_v[pl.ds(0, 16)]
    for k in range(_AHEAD):
        issue(uc0[k], fl0[k], sl0[k])

    inflight_i = iprep(0, 0)
    for j in range(_NCHUNK):
        b = j % 2
        cur_i = inflight_i
        if j + 1 < _NCHUNK:
            inflight_i = iprep(j + 1, 1 - b)
        cur_i.wait()

        def block16(g, _):
            rbase = j * _CHUNK + g * 16
            uc = su_v[pl.ds(rbase, 16)]
            flc = fl_v[pl.ds(rbase, 16)]
            slc = sl_v[pl.ds(rbase, 16)]
            ipar = (si_v[pl.ds(rbase, 16)] & 1) * EMBED_DIM
            nxt = jnp.minimum(rbase + 16, _BPW - 16)
            un = su_v[pl.ds(nxt, 16)]
            fln = fl_v[pl.ds(nxt, 16)]
            sln = sl_v[pl.ds(nxt, 16)]
            acc = jnp.zeros((16,), jnp.float32)
            for k in range(16):
                wait_for(flc[k], slc[k])
                lane = jnp.broadcast_to(uc[k] & 127, (16,))
                srow = slc[k] * EMBED_DIM
                psum = jnp.zeros((16,), jnp.float32)
                for q in range(EMBED_DIM // 16):
                    u = plsc.load_gather(ublk, [srow + d16[q], lane])
                    v = ibuf[b * _CHUNK + g * 16 + k,
                             pl.ds(ipar[k] + 16 * q, 16)]
                    psum = psum + u * v
                if k < _AHEAD:
                    issue(uc[k + _AHEAD], flc[k + _AHEAD], slc[k + _AHEAD])
                else:
                    kk = k - _AHEAD

                    @pl.when(rbase + 16 + kk < _BPW)
                    def _():
                        issue(un[kk], fln[kk], sln[kk])
                acc = jnp.where(lanes == k, jnp.sum(psum), acc)
            out_v[pl.ds(rbase, 16)] = acc
            return 0

        lax.fori_loop(0, _CHUNK // 16, block16, 0)

    pltpu.sync_copy(out_v, out_hbm.at[pl.ds(base, _BPW)])


@jax.jit
def _cf_kernel(user_id, item_id, user_table, item_table):
    mesh = plsc.VectorSubcoreMesh(core_axis_name="c", subcore_axis_name="s")
    f = pl.kernel(
        _cf_body,
        out_type=jax.ShapeDtypeStruct((BATCH,), jnp.float32),
        mesh=mesh,
        scratch_types=[
            pltpu.VMEM((_BPW,), jnp.int32),
            pltpu.VMEM((_BPW,), jnp.int32),
            pltpu.VMEM((_BPW,), jnp.int32),
            pltpu.VMEM((_BPW,), jnp.int32),
            pltpu.VMEM((2 * _CHUNK,), jnp.int32),
            pltpu.VMEM((_NRING * EMBED_DIM, 128), jnp.float32),
            pltpu.VMEM((2 * _CHUNK, 2 * EMBED_DIM), jnp.float32),
            pltpu.VMEM((_BPW,), jnp.float32),
            pltpu.SemaphoreType.DMA((_NRING,)),
            pltpu.SemaphoreType.DMA((2,)),
        ],
        compiler_params=pltpu.CompilerParams(
            needs_layout_passes=False, use_tc_tiling_on_sc=True),
    )
    order = jnp.argsort(user_id)
    su = user_id[order]
    si = item_id[order]
    blk = su >> 7
    prev = jnp.concatenate([blk[:1] - 1, blk[:-1]])
    pos = jnp.arange(BATCH, dtype=jnp.int32)
    flags = ((pos % _BPW == 0) | (blk != prev)).astype(jnp.int32)
    slots = (jnp.cumsum(flags) - 1).astype(jnp.int32) % _NRING
    it2 = item_table.reshape(item_table.shape[0] // 2, 2 * EMBED_DIM)
    out_sorted = f(su, si, flags, slots, user_table.T, it2)
    inv = jnp.zeros((BATCH,), jnp.int32).at[order].set(pos)
    return out_sorted[inv]


def kernel(user_id, item_id, user_table, item_table):
    out = _cf_kernel(user_id, item_id, user_table, item_table)
    return out.reshape(BATCH, 1)


# speedup vs baseline: 1.3244x; 1.0468x over previous
import functools

import jax
import jax.numpy as jnp
from jax import lax
from jax.experimental import pallas as pl
from jax.experimental.pallas import tpu as pltpu
from jax.experimental.pallas import tpu_sc as plsc

BATCH = 16384
EMBED_DIM = 64
_NC = 2
_NS = 16
_NW = _NC * _NS
_BPW = BATCH // _NW
_CHUNK = 128
_NCHUNK = _BPW // _CHUNK
_NRING = 9
_AHEAD = 8


def _cf_body(su_hbm, ord_hbm, iid_hbm, fl_hbm, sl_hbm, ut_hbm, it_hbm,
             out_hbm, su_v, ord_v, iid_all, si_v, fl_v, sl_v, ipair, ublk,
             ibuf, out_v, usem, isem):
    wid = lax.axis_index("s") * _NC + lax.axis_index("c")
    base = wid * _BPW

    pltpu.sync_copy(su_hbm.at[pl.ds(base, _BPW)], su_v)
    pltpu.sync_copy(ord_hbm.at[pl.ds(base, _BPW)], ord_v)
    pltpu.sync_copy(iid_hbm, iid_all)
    pltpu.sync_copy(fl_hbm.at[pl.ds(base, _BPW)], fl_v)
    pltpu.sync_copy(sl_hbm.at[pl.ds(base, _BPW)], sl_v)

    def iprep(j, b):
        for t in range(_CHUNK // 16):
            s = pl.ds(j * _CHUNK + t * 16, 16)
            si16 = plsc.load_gather(iid_all, [ord_v[s]])
            si_v.at[s][...] = si16
            ipair.at[pl.ds(b * _CHUNK + t * 16, 16)][...] = si16 >> 1
        return pltpu.async_copy(
            it_hbm.at[ipair.at[pl.ds(b * _CHUNK, _CHUNK)]],
            ibuf.at[pl.ds(b * _CHUNK, _CHUNK), :], isem.at[b])

    def issue(uid, flag, slot):
        @pl.when(flag == 1)
        def _():
            blk = pl.multiple_of((uid >> 7) * 128, 128)
            row = pl.multiple_of(slot * EMBED_DIM, 8)
            pltpu.async_copy(ut_hbm.at[:, pl.ds(blk, 128)],
                             ublk.at[pl.ds(row, EMBED_DIM), :], usem.at[slot])

    def wait_for(flag, slot):
        @pl.when(flag == 1)
        def _():
            row = pl.multiple_of(slot * EMBED_DIM, 8)
            pltpu.make_async_copy(ut_hbm.at[:, pl.ds(0, 128)],
                                  ublk.at[pl.ds(row, EMBED_DIM), :],
                                  usem.at[slot]).wait()

    lanes = lax.iota(jnp.int32, 16)
    d16 = [lax.iota(jnp.int32, 16) + 16 * q for q in range(EMBED_DIM // 16)]

    uc0 = su_v[pl.ds(0, 16)]
    fl0 = fl_v[pl.ds(0, 16)]
    sl0 = sl_v[pl.ds(0, 16)]
    for k in range(_AHEAD):
        issue(uc0[k], fl0[k], sl0[k])

    inflight_i = iprep(0, 0)
    for j in range(_NCHUNK):
        b = j % 2
        cur_i = inflight_i
        if j + 1 < _NCHUNK:
            inflight_i = iprep(j + 1, 1 - b)
        cur_i.wait()

        def block16(g, _):
            rbase = j * _CHUNK + g * 16
            uc = su_v[pl.ds(rbase, 16)]
            flc = fl_v[pl.ds(rbase, 16)]
            slc = sl_v[pl.ds(rbase, 16)]
            ipar = (si_v[pl.ds(rbase, 16)] & 1) * EMBED_DIM
            nxt = jnp.minimum(rbase + 16, _BPW - 16)
            un = su_v[pl.ds(nxt, 16)]
            fln = fl_v[pl.ds(nxt, 16)]
            sln = sl_v[pl.ds(nxt, 16)]
            acc = jnp.zeros((16,), jnp.float32)
            for k in range(16):
                wait_for(flc[k], slc[k])
                lane = jnp.broadcast_to(uc[k] & 127, (16,))
                srow = slc[k] * EMBED_DIM
                psum = jnp.zeros((16,), jnp.float32)
                for q in range(EMBED_DIM // 16):
                    u = plsc.load_gather(ublk, [srow + d16[q], lane])
                    v = ibuf[b * _CHUNK + g * 16 + k,
                             pl.ds(ipar[k] + 16 * q, 16)]
                    psum = psum + u * v
                if k < _AHEAD:
                    issue(uc[k + _AHEAD], flc[k + _AHEAD], slc[k + _AHEAD])
                else:
                    kk = k - _AHEAD

                    @pl.when(rbase + 16 + kk < _BPW)
                    def _():
                        issue(un[kk], fln[kk], sln[kk])
                acc = jnp.where(lanes == k, jnp.sum(psum), acc)
            out_v[pl.ds(rbase, 16)] = acc
            return 0

        lax.fori_loop(0, _CHUNK // 16, block16, 0)

    pltpu.sync_copy(out_v, out_hbm.at[pl.ds(base, _BPW)])


@jax.jit
def _cf_kernel(user_id, item_id, user_table, item_table):
    mesh = plsc.VectorSubcoreMesh(core_axis_name="c", subcore_axis_name="s")
    f = pl.kernel(
        _cf_body,
        out_type=jax.ShapeDtypeStruct((BATCH,), jnp.float32),
        mesh=mesh,
        scratch_types=[
            pltpu.VMEM((_BPW,), jnp.int32),
            pltpu.VMEM((_BPW,), jnp.int32),
            pltpu.VMEM((BATCH,), jnp.int32),
            pltpu.VMEM((_BPW,), jnp.int32),
            pltpu.VMEM((_BPW,), jnp.int32),
            pltpu.VMEM((_BPW,), jnp.int32),
            pltpu.VMEM((2 * _CHUNK,), jnp.int32),
            pltpu.VMEM((_NRING * EMBED_DIM, 128), jnp.float32),
            pltpu.VMEM((2 * _CHUNK, 2 * EMBED_DIM), jnp.float32),
            pltpu.VMEM((_BPW,), jnp.float32),
            pltpu.SemaphoreType.DMA((_NRING,)),
            pltpu.SemaphoreType.DMA((2,)),
        ],
        compiler_params=pltpu.CompilerParams(
            needs_layout_passes=False, use_tc_tiling_on_sc=True),
    )
    pos = jnp.arange(BATCH, dtype=jnp.int32)
    su, order = lax.sort((user_id, pos), num_keys=1)
    blk = su >> 7
    prev = jnp.concatenate([blk[:1] - 1, blk[:-1]])
    flags = ((pos % _BPW == 0) | (blk != prev)).astype(jnp.int32)
    slots = (jnp.cumsum(flags) - 1).astype(jnp.int32) % _NRING
    it2 = item_table.reshape(item_table.shape[0] // 2, 2 * EMBED_DIM)
    out_sorted = f(su, order, item_id, flags, slots, user_table.T, it2)
    inv = jnp.zeros((BATCH,), jnp.int32).at[order].set(pos)
    return out_sorted[inv]


def kernel(user_id, item_id, user_table, item_table):
    out = _cf_kernel(user_id, item_id, user_table, item_table)
    return out.reshape(BATCH, 1)
